# baseline (speedup 1.0000x reference)
import math

import jax
import jax.numpy as jnp
import numpy as np
from jax.experimental import pallas as pl
from jax.experimental.pallas import tpu as pltpu

_N = 16384
_D = 128
_UNROLL = 32
_XROWS = _N // _D


def _pe_kernel(x_ref, f_ref, p_ref, b_ref, out_ref):
    fcol = f_ref[...]
    p = p_ref[...]
    bias = b_ref[...]
    def body(u, carry):
        xrow = x_ref[pl.ds(u, 1), :]
        arg = fcol * xrow
        s = jnp.sin(arg)
        c = jnp.cos(arg)
        sc = jnp.concatenate([s, c], axis=0)
        out = jax.lax.dot_general(
            sc,
            p,
            (((0,), (0,)), ((), ())),
            preferred_element_type=jnp.float32,
            precision=jax.lax.Precision.HIGHEST,
        )
        out_ref[pl.ds(u * _D, _D), :] = out + bias
        return carry

    jax.lax.fori_loop(0, _UNROLL, body, 0, unroll=2)


def kernel(x, E_class):
    del E_class
    x2 = x.reshape(_XROWS, _D)
    angles = jnp.arange(_D, dtype=jnp.float32)
    freqs = (jnp.power(2.0, angles) * math.pi).reshape(_D, 1)
    freqs = freqs.at[_D - 1, 0].set(0.0)
    perm = np.zeros((2 * _D, 2 * _D), np.float32)
    kk = np.arange(_D)
    perm[kk, 2 * kk] = 1.0
    perm[_D + kk, 2 * kk + 1] = 1.0
    bias = np.zeros((1, 2 * _D), np.float32)
    bias[0, 2 * _D - 2 :] = np.nan
    grid = (_XROWS // _UNROLL,)
    return pl.pallas_call(
        _pe_kernel,
        grid=grid,
        in_specs=[
            pl.BlockSpec((_UNROLL, _D), lambda i: (i, 0)),
            pl.BlockSpec((_D, 1), lambda i: (0, 0)),
            pl.BlockSpec((2 * _D, 2 * _D), lambda i: (0, 0)),
            pl.BlockSpec((1, 2 * _D), lambda i: (0, 0)),
        ],
        out_specs=pl.BlockSpec((_UNROLL * _D, 2 * _D), lambda i: (i, 0)),
        out_shape=jax.ShapeDtypeStruct((_N, 2 * _D), jnp.float32),
        compiler_params=pltpu.CompilerParams(
            dimension_semantics=("parallel",),
        ),
    )(x2, freqs, perm, bias)


# speedup vs baseline: 1.4237x; 1.4237x over previous
import math

import jax
import jax.numpy as jnp
import numpy as np
from jax.experimental import pallas as pl
from jax.experimental.pallas import tpu as pltpu

_N = 16384
_D = 128
_UNROLL = 32
_XROWS = _N // _D


def _pe_kernel(x_ref, f_ref, p_ref, out_ref):
    fcol = f_ref[...]
    p = p_ref[...]
    for u in range(_UNROLL):
        xrow = x_ref[pl.ds(u, 1), :]
        arg = fcol * xrow
        s = jnp.sin(arg)
        c = jnp.cos(arg)
        sc = jnp.concatenate([s, c], axis=0)
        out = jax.lax.dot_general(
            sc,
            p,
            (((0,), (0,)), ((), ())),
            preferred_element_type=jnp.float32,
            precision=jax.lax.Precision.HIGHEST,
        )
        out_ref[pl.ds(u * _D, _D), :] = out


def kernel(x, E_class):
    del E_class
    x2 = x.reshape(_XROWS, _D)
    angles = jnp.arange(_D, dtype=jnp.float32)
    freqs = (jnp.power(2.0, angles) * math.pi).reshape(_D, 1)
    freqs = freqs.at[_D - 1, 0].set(0.0)
    perm = np.zeros((2 * _D, 2 * _D), np.float32)
    kk = np.arange(_D)
    perm[kk, 2 * kk] = 1.0
    perm[_D + kk, 2 * kk + 1] = 1.0
    perm[_D - 1, 2 * _D - 2] = np.nan
    perm[2 * _D - 1, 2 * _D - 1] = np.nan
    grid = (_XROWS // _UNROLL,)
    return pl.pallas_call(
        _pe_kernel,
        grid=grid,
        in_specs=[
            pl.BlockSpec((_UNROLL, _D), lambda i: (i, 0)),
            pl.BlockSpec((_D, 1), lambda i: (0, 0)),
            pl.BlockSpec((2 * _D, 2 * _D), lambda i: (0, 0)),
        ],
        out_specs=pl.BlockSpec((_UNROLL * _D, 2 * _D), lambda i: (i, 0)),
        out_shape=jax.ShapeDtypeStruct((_N, 2 * _D), jnp.float32),
        compiler_params=pltpu.CompilerParams(
            dimension_semantics=("parallel",),
        ),
    )(x2, freqs, perm)
